# baseline (device time: 33973 ns/iter reference)
import jax
import jax.numpy as jnp
from jax import lax
from jax.experimental import pallas as pl
from jax.experimental.pallas import tpu as pltpu

N_DEV = 4
B = 2
SQ = 128
SKV = 128
D = 512
HQ = 8
DH = 64
SCALE = 0.125


def kernel(x, Wq, Wo, K_ext, V_ext):
    pos = lax.axis_index("i")
    K_loc = lax.dynamic_slice_in_dim(K_ext, pos * HQ, HQ, axis=2)
    V_loc = lax.dynamic_slice_in_dim(V_ext, pos * HQ, HQ, axis=2)
    K3 = K_loc.transpose(0, 2, 1, 3).astype(jnp.bfloat16)
    V3 = V_loc.transpose(0, 2, 1, 3).astype(jnp.bfloat16)
    x2 = x.reshape(B * SQ, D).astype(jnp.bfloat16)
    Wq_b = Wq.astype(jnp.bfloat16)
    Wo_b = Wo.astype(jnp.bfloat16)

    def body(x_ref, wq_ref, wo_ref, k_ref, v_ref, out_ref,
             attn_ref, comm_ref, send_sems, recv_sems):
        my_pos = lax.axis_index("i")
        left = lax.rem(my_pos + N_DEV - 1, N_DEV)
        right = lax.rem(my_pos + 1, N_DEV)

        barrier_sem = pltpu.get_barrier_semaphore()
        for nbr in (left, right):
            pl.semaphore_signal(
                barrier_sem, inc=1,
                device_id=(nbr,), device_id_type=pl.DeviceIdType.MESH,
            )
        pl.semaphore_wait(barrier_sem, 2)

        q = lax.dot(x_ref[...], wq_ref[...],
                    preferred_element_type=jnp.float32)
        q = q.astype(jnp.bfloat16)

        for b in range(B):
            for h in range(HQ):
                qbh = q[b * SQ:(b + 1) * SQ, h * DH:(h + 1) * DH]
                kbh = k_ref[b, h]
                vbh = v_ref[b, h]
                s = lax.dot_general(
                    qbh, kbh, (((1,), (1,)), ((), ())),
                    preferred_element_type=jnp.float32) * SCALE
                m = jnp.max(s, axis=1, keepdims=True)
                p = jnp.exp(s - m)
                l = jnp.sum(p, axis=1, keepdims=True)
                o = lax.dot(p.astype(jnp.bfloat16), vbh,
                            preferred_element_type=jnp.float32)
                attn_ref[b * SQ:(b + 1) * SQ, h * DH:(h + 1) * DH] = (
                    (o / l).astype(jnp.bfloat16))

        partial = lax.dot(attn_ref[...], wo_ref[...],
                          preferred_element_type=jnp.float32)

        out_ref[...] = partial
        comm_ref[0] = partial

        for hop in range(N_DEV - 1):
            rdma = pltpu.make_async_remote_copy(
                src_ref=comm_ref.at[hop],
                dst_ref=comm_ref.at[hop + 1],
                send_sem=send_sems.at[hop],
                recv_sem=recv_sems.at[hop],
                device_id=(right,),
                device_id_type=pl.DeviceIdType.MESH,
            )
            rdma.start()
            rdma.wait()
            out_ref[...] += comm_ref[hop + 1]

    out2 = pl.pallas_call(
        body,
        out_shape=jax.ShapeDtypeStruct((B * SQ, D), jnp.float32),
        in_specs=[pl.BlockSpec(memory_space=pltpu.VMEM)] * 5,
        out_specs=pl.BlockSpec(memory_space=pltpu.VMEM),
        scratch_shapes=[
            pltpu.VMEM((B * SQ, HQ * DH), jnp.bfloat16),
            pltpu.VMEM((N_DEV, B * SQ, D), jnp.float32),
            pltpu.SemaphoreType.DMA((N_DEV - 1,)),
            pltpu.SemaphoreType.DMA((N_DEV - 1,)),
        ],
        compiler_params=pltpu.CompilerParams(collective_id=0),
    )(x2, Wq_b, Wo_b, K3, V3)
    return out2.reshape(B, SQ, D)


# device time: 19411 ns/iter; 1.7502x vs baseline; 1.7502x over previous
import jax
import jax.numpy as jnp
from jax import lax
from jax.experimental import pallas as pl
from jax.experimental.pallas import tpu as pltpu

N_DEV = 4
B = 2
SQ = 128
SKV = 128
D = 512
HQ = 8
DH = 64
SCALE = 0.125


def kernel(x, Wq, Wo, K_ext, V_ext):
    pos = lax.axis_index("i")
    K_loc = lax.dynamic_slice_in_dim(K_ext, pos * HQ, HQ, axis=2)
    V_loc = lax.dynamic_slice_in_dim(V_ext, pos * HQ, HQ, axis=2)
    K3 = K_loc.transpose(0, 2, 1, 3).astype(jnp.bfloat16)
    V3 = V_loc.transpose(0, 2, 1, 3).astype(jnp.bfloat16)
    x2 = x.reshape(B * SQ, D).astype(jnp.bfloat16)
    Wq_b = Wq.astype(jnp.bfloat16)
    Wo_b = Wo.astype(jnp.bfloat16)

    def body(x_ref, wq_ref, wo_ref, k_ref, v_ref, out_ref,
             attn_ref, comm_ref, send_sems, recv_sems):
        my_pos = lax.axis_index("i")

        barrier_sem = pltpu.get_barrier_semaphore()
        for d in range(1, N_DEV):
            peer = lax.rem(my_pos + d, N_DEV)
            pl.semaphore_signal(
                barrier_sem, inc=1,
                device_id=(peer,), device_id_type=pl.DeviceIdType.MESH,
            )
        pl.semaphore_wait(barrier_sem, N_DEV - 1)

        q = lax.dot(x_ref[...], wq_ref[...],
                    preferred_element_type=jnp.float32)
        q = q.astype(jnp.bfloat16)

        for b in range(B):
            for h in range(HQ):
                qbh = q[b * SQ:(b + 1) * SQ, h * DH:(h + 1) * DH]
                kbh = k_ref[b, h]
                vbh = v_ref[b, h]
                s = lax.dot_general(
                    qbh, kbh, (((1,), (1,)), ((), ())),
                    preferred_element_type=jnp.float32) * SCALE
                m = jnp.max(s, axis=1, keepdims=True)
                p = jnp.exp(s - m)
                l = jnp.sum(p, axis=1, keepdims=True)
                o = lax.dot(p.astype(jnp.bfloat16), vbh,
                            preferred_element_type=jnp.float32)
                attn_ref[b * SQ:(b + 1) * SQ, h * DH:(h + 1) * DH] = (
                    (o / l).astype(jnp.bfloat16))

        partial = lax.dot(attn_ref[...], wo_ref[...],
                          preferred_element_type=jnp.float32)

        out_ref[...] = partial
        comm_ref[0] = partial.astype(jnp.bfloat16)

        rdmas = []
        for d in range(1, N_DEV):
            peer = lax.rem(my_pos + d, N_DEV)
            rdma = pltpu.make_async_remote_copy(
                src_ref=comm_ref.at[0],
                dst_ref=comm_ref.at[d],
                send_sem=send_sems.at[d - 1],
                recv_sem=recv_sems.at[d - 1],
                device_id=(peer,),
                device_id_type=pl.DeviceIdType.MESH,
            )
            rdma.start()
            rdmas.append(rdma)
        for rdma in rdmas:
            rdma.wait_recv()
        out_ref[...] += (
            comm_ref[1].astype(jnp.float32)
            + comm_ref[2].astype(jnp.float32)
            + comm_ref[3].astype(jnp.float32)
        )
        for rdma in rdmas:
            rdma.wait_send()

    out2 = pl.pallas_call(
        body,
        out_shape=jax.ShapeDtypeStruct((B * SQ, D), jnp.float32),
        in_specs=[pl.BlockSpec(memory_space=pltpu.VMEM)] * 5,
        out_specs=pl.BlockSpec(memory_space=pltpu.VMEM),
        scratch_shapes=[
            pltpu.VMEM((B * SQ, HQ * DH), jnp.bfloat16),
            pltpu.VMEM((N_DEV, B * SQ, D), jnp.bfloat16),
            pltpu.SemaphoreType.DMA((N_DEV - 1,)),
            pltpu.SemaphoreType.DMA((N_DEV - 1,)),
        ],
        compiler_params=pltpu.CompilerParams(collective_id=0),
    )(x2, Wq_b, Wo_b, K3, V3)
    return out2.reshape(B, SQ, D)


# device time: 17084 ns/iter; 1.9886x vs baseline; 1.1362x over previous
import jax
import jax.numpy as jnp
from jax import lax
from jax.experimental import pallas as pl
from jax.experimental.pallas import tpu as pltpu

N_DEV = 4
B = 2
SQ = 128
SKV = 128
D = 512
HQ = 8
DH = 64
SCALE = 0.125


def kernel(x, Wq, Wo, K_ext, V_ext):
    pos = lax.axis_index("i")
    K2 = lax.dynamic_slice_in_dim(
        K_ext.reshape(B, SKV, 4 * HQ * DH), pos * HQ * DH, HQ * DH, axis=2
    ).astype(jnp.bfloat16)
    V2 = lax.dynamic_slice_in_dim(
        V_ext.reshape(B, SKV, 4 * HQ * DH), pos * HQ * DH, HQ * DH, axis=2
    ).astype(jnp.bfloat16)
    x2 = x.reshape(B * SQ, D).astype(jnp.bfloat16)
    Wq_b = Wq.astype(jnp.bfloat16)
    Wo_b = Wo.astype(jnp.bfloat16)

    def body(x_ref, wq_ref, wo_ref, k_ref, v_ref, out_ref,
             attn_ref, comm_ref, send_sems, recv_sems):
        my_pos = lax.axis_index("i")

        barrier_sem = pltpu.get_barrier_semaphore()
        for d in range(1, N_DEV):
            peer = lax.rem(my_pos + d, N_DEV)
            pl.semaphore_signal(
                barrier_sem, inc=1,
                device_id=(peer,), device_id_type=pl.DeviceIdType.MESH,
            )
        pl.semaphore_wait(barrier_sem, N_DEV - 1)

        q = lax.dot(x_ref[...], wq_ref[...],
                    preferred_element_type=jnp.float32)
        q = q.astype(jnp.bfloat16)

        rdmas = []
        for b in range(B):
            for h in range(HQ):
                qbh = q[b * SQ:(b + 1) * SQ, h * DH:(h + 1) * DH]
                kbh = k_ref[b][:, h * DH:(h + 1) * DH]
                vbh = v_ref[b][:, h * DH:(h + 1) * DH]
                s = lax.dot_general(
                    qbh, kbh, (((1,), (1,)), ((), ())),
                    preferred_element_type=jnp.float32) * SCALE
                m = jnp.max(s, axis=1, keepdims=True)
                p = jnp.exp(s - m)
                l = jnp.sum(p, axis=1, keepdims=True)
                o = lax.dot(p.astype(jnp.bfloat16), vbh,
                            preferred_element_type=jnp.float32)
                attn_ref[b * SQ:(b + 1) * SQ, h * DH:(h + 1) * DH] = (
                    (o / l).astype(jnp.bfloat16))

            partial_b = lax.dot(
                attn_ref[b * SQ:(b + 1) * SQ, :], wo_ref[...],
                preferred_element_type=jnp.float32)
            out_ref[b * SQ:(b + 1) * SQ, :] = partial_b
            comm_ref[0, b] = partial_b.astype(jnp.bfloat16)

            for d in range(1, N_DEV):
                peer = lax.rem(my_pos + d, N_DEV)
                rdma = pltpu.make_async_remote_copy(
                    src_ref=comm_ref.at[0, b],
                    dst_ref=comm_ref.at[d, b],
                    send_sem=send_sems.at[d - 1, b],
                    recv_sem=recv_sems.at[d - 1, b],
                    device_id=(peer,),
                    device_id_type=pl.DeviceIdType.MESH,
                )
                rdma.start()
                rdmas.append(rdma)

        for rdma in rdmas:
            rdma.wait_recv()
        for b in range(B):
            out_ref[b * SQ:(b + 1) * SQ, :] += (
                comm_ref[1, b].astype(jnp.float32)
                + comm_ref[2, b].astype(jnp.float32)
                + comm_ref[3, b].astype(jnp.float32)
            )
        for rdma in rdmas:
            rdma.wait_send()

    out2 = pl.pallas_call(
        body,
        out_shape=jax.ShapeDtypeStruct((B * SQ, D), jnp.float32),
        in_specs=[pl.BlockSpec(memory_space=pltpu.VMEM)] * 5,
        out_specs=pl.BlockSpec(memory_space=pltpu.VMEM),
        scratch_shapes=[
            pltpu.VMEM((B * SQ, HQ * DH), jnp.bfloat16),
            pltpu.VMEM((N_DEV, B, SQ, D), jnp.bfloat16),
            pltpu.SemaphoreType.DMA((N_DEV - 1, B)),
            pltpu.SemaphoreType.DMA((N_DEV - 1, B)),
        ],
        compiler_params=pltpu.CompilerParams(collective_id=0),
    )(x2, Wq_b, Wo_b, K2, V2)
    return out2.reshape(B, SQ, D)


# device time: 16864 ns/iter; 2.0145x vs baseline; 1.0130x over previous
import jax
import jax.numpy as jnp
from jax import lax
from jax.experimental import pallas as pl
from jax.experimental.pallas import tpu as pltpu

N_DEV = 4
B = 2
SQ = 128
SKV = 128
D = 512
HQ = 8
DH = 64
SCALE = 0.125
BF = jnp.bfloat16


def kernel(x, Wq, Wo, K_ext, V_ext):
    K2 = K_ext.reshape(B, SKV, 4 * HQ * DH)
    V2 = V_ext.reshape(B, SKV, 4 * HQ * DH)

    def body(x_ref, wq_ref, wo_ref, k_hbm, v_hbm, out_ref,
             k_ref, v_ref, attn_ref, comm_ref,
             kv_sems, send_sems, recv_sems):
        my_pos = lax.axis_index("i")

        barrier_sem = pltpu.get_barrier_semaphore()
        for d in range(1, N_DEV):
            peer = lax.rem(my_pos + d, N_DEV)
            pl.semaphore_signal(
                barrier_sem, inc=1,
                device_id=(peer,), device_id_type=pl.DeviceIdType.MESH,
            )

        kcopy = pltpu.make_async_copy(
            k_hbm.at[:, :, pl.ds(my_pos * HQ * DH, HQ * DH)],
            k_ref, kv_sems.at[0])
        vcopy = pltpu.make_async_copy(
            v_hbm.at[:, :, pl.ds(my_pos * HQ * DH, HQ * DH)],
            v_ref, kv_sems.at[1])
        kcopy.start()
        vcopy.start()

        wqb = wq_ref[...].astype(BF)
        q0 = lax.dot(x_ref[0].astype(BF), wqb,
                     preferred_element_type=jnp.float32).astype(BF)
        q1 = lax.dot(x_ref[1].astype(BF), wqb,
                     preferred_element_type=jnp.float32).astype(BF)
        qs = (q0, q1)
        wob = wo_ref[...].astype(BF)

        kcopy.wait()
        vcopy.wait()

        rdmas = []
        for b in range(B):
            kb = k_ref[b].astype(BF)
            vb = v_ref[b].astype(BF)
            for h in range(HQ):
                qbh = qs[b][:, h * DH:(h + 1) * DH]
                kbh = kb[:, h * DH:(h + 1) * DH]
                vbh = vb[:, h * DH:(h + 1) * DH]
                s = lax.dot_general(
                    qbh, kbh, (((1,), (1,)), ((), ())),
                    preferred_element_type=jnp.float32) * SCALE
                m = jnp.max(s, axis=1, keepdims=True)
                p = jnp.exp(s - m)
                l = jnp.sum(p, axis=1, keepdims=True)
                o = lax.dot(p.astype(BF), vbh,
                            preferred_element_type=jnp.float32)
                attn_ref[b * SQ:(b + 1) * SQ, h * DH:(h + 1) * DH] = (
                    (o / l).astype(BF))

            partial_b = lax.dot(
                attn_ref[b * SQ:(b + 1) * SQ, :], wob,
                preferred_element_type=jnp.float32)
            out_ref[b] = partial_b
            comm_ref[0, b] = partial_b.astype(BF)

            if b == 0:
                pl.semaphore_wait(barrier_sem, N_DEV - 1)

            for d in range(1, N_DEV):
                peer = lax.rem(my_pos + d, N_DEV)
                rdma = pltpu.make_async_remote_copy(
                    src_ref=comm_ref.at[0, b],
                    dst_ref=comm_ref.at[d, b],
                    send_sem=send_sems.at[d - 1, b],
                    recv_sem=recv_sems.at[d - 1, b],
                    device_id=(peer,),
                    device_id_type=pl.DeviceIdType.MESH,
                )
                rdma.start()
                rdmas.append(rdma)

        for rdma in rdmas:
            rdma.wait_recv()
        for b in range(B):
            out_ref[b] += (
                comm_ref[1, b].astype(jnp.float32)
                + comm_ref[2, b].astype(jnp.float32)
                + comm_ref[3, b].astype(jnp.float32)
            )
        for rdma in rdmas:
            rdma.wait_send()

    return pl.pallas_call(
        body,
        out_shape=jax.ShapeDtypeStruct((B, SQ, D), jnp.float32),
        in_specs=[
            pl.BlockSpec(memory_space=pltpu.VMEM),
            pl.BlockSpec(memory_space=pltpu.VMEM),
            pl.BlockSpec(memory_space=pltpu.VMEM),
            pl.BlockSpec(memory_space=pl.ANY),
            pl.BlockSpec(memory_space=pl.ANY),
        ],
        out_specs=pl.BlockSpec(memory_space=pltpu.VMEM),
        scratch_shapes=[
            pltpu.VMEM((B, SKV, HQ * DH), jnp.float32),
            pltpu.VMEM((B, SKV, HQ * DH), jnp.float32),
            pltpu.VMEM((B * SQ, HQ * DH), BF),
            pltpu.VMEM((N_DEV, B, SQ, D), BF),
            pltpu.SemaphoreType.DMA((2,)),
            pltpu.SemaphoreType.DMA((N_DEV - 1, B)),
            pltpu.SemaphoreType.DMA((N_DEV - 1, B)),
        ],
        compiler_params=pltpu.CompilerParams(collective_id=0),
    )(x, Wq, Wo, K2, V2)


# device time: 15636 ns/iter; 2.1727x vs baseline; 1.0785x over previous
import jax
import jax.numpy as jnp
from jax import lax
from jax.experimental import pallas as pl
from jax.experimental.pallas import tpu as pltpu

N_DEV = 4
B = 2
SQ = 128
SKV = 128
D = 512
HQ = 8
DH = 64
SCALE = 0.125
BF = jnp.bfloat16


def kernel(x, Wq, Wo, K_ext, V_ext):
    pos = lax.axis_index("i")
    K2 = lax.dynamic_slice_in_dim(
        K_ext.reshape(B, SKV, 4 * HQ * DH), pos * HQ * DH, HQ * DH, axis=2
    ).astype(BF)
    V2 = lax.dynamic_slice_in_dim(
        V_ext.reshape(B, SKV, 4 * HQ * DH), pos * HQ * DH, HQ * DH, axis=2
    ).astype(BF)

    def body(x_ref, wq_ref, wo_ref, k_ref, v_ref, out_ref,
             attn_ref, comm_ref, send_sems, recv_sems):
        my_pos = lax.axis_index("i")

        barrier_sem = pltpu.get_barrier_semaphore()
        for d in range(1, N_DEV):
            peer = lax.rem(my_pos + d, N_DEV)
            pl.semaphore_signal(
                barrier_sem, inc=1,
                device_id=(peer,), device_id_type=pl.DeviceIdType.MESH,
            )

        wqb = wq_ref[...].astype(BF)
        q0 = lax.dot(x_ref[0].astype(BF), wqb,
                     preferred_element_type=jnp.float32).astype(BF)
        q1 = lax.dot(x_ref[1].astype(BF), wqb,
                     preferred_element_type=jnp.float32).astype(BF)
        qs = (q0, q1)
        wob = wo_ref[...].astype(BF)

        rdmas = []
        for b in range(B):
            kb = k_ref[b]
            vb = v_ref[b]
            for h in range(HQ):
                qbh = qs[b][:, h * DH:(h + 1) * DH]
                kbh = kb[:, h * DH:(h + 1) * DH]
                vbh = vb[:, h * DH:(h + 1) * DH]
                s = lax.dot_general(
                    qbh, kbh, (((1,), (1,)), ((), ())),
                    preferred_element_type=jnp.float32) * SCALE
                m = jnp.max(s, axis=1, keepdims=True)
                p = jnp.exp(s - m)
                l = jnp.sum(p, axis=1, keepdims=True)
                o = lax.dot(p.astype(BF), vbh,
                            preferred_element_type=jnp.float32)
                attn_ref[b * SQ:(b + 1) * SQ, h * DH:(h + 1) * DH] = (
                    (o / l).astype(BF))

            partial_b = lax.dot(
                attn_ref[b * SQ:(b + 1) * SQ, :], wob,
                preferred_element_type=jnp.float32)
            out_ref[b] = partial_b
            comm_ref[0, b] = partial_b.astype(BF)

            if b == 0:
                pl.semaphore_wait(barrier_sem, N_DEV - 1)

            for d in range(1, N_DEV):
                peer = lax.rem(my_pos + d, N_DEV)
                rdma = pltpu.make_async_remote_copy(
                    src_ref=comm_ref.at[0, b],
                    dst_ref=comm_ref.at[d, b],
                    send_sem=send_sems.at[d - 1, b],
                    recv_sem=recv_sems.at[d - 1, b],
                    device_id=(peer,),
                    device_id_type=pl.DeviceIdType.MESH,
                )
                rdma.start()
                rdmas.append(rdma)

        for rdma in rdmas:
            rdma.wait_recv()
        for b in range(B):
            out_ref[b] += (
                comm_ref[1, b].astype(jnp.float32)
                + comm_ref[2, b].astype(jnp.float32)
                + comm_ref[3, b].astype(jnp.float32)
            )
        for rdma in rdmas:
            rdma.wait_send()

    return pl.pallas_call(
        body,
        out_shape=jax.ShapeDtypeStruct((B, SQ, D), jnp.float32),
        in_specs=[pl.BlockSpec(memory_space=pltpu.VMEM)] * 5,
        out_specs=pl.BlockSpec(memory_space=pltpu.VMEM),
        scratch_shapes=[
            pltpu.VMEM((B * SQ, HQ * DH), BF),
            pltpu.VMEM((N_DEV, B, SQ, D), BF),
            pltpu.SemaphoreType.DMA((N_DEV - 1, B)),
            pltpu.SemaphoreType.DMA((N_DEV - 1, B)),
        ],
        compiler_params=pltpu.CompilerParams(collective_id=0),
    )(x, Wq, Wo, K2, V2)
